# Initial kernel scaffold; baseline (speedup 1.0000x reference)
import jax
import jax.numpy as jnp
from jax.experimental import pallas as pl


def kernel(x, vocab_table):
    raise NotImplementedError("write your pallas kernel here")


# speedup vs baseline: 1.1801x; 1.1801x over previous
import functools

import jax
import jax.numpy as jnp
from jax import lax
from jax.experimental import pallas as pl
from jax.experimental.pallas import tpu as pltpu
from jax.experimental.pallas import tpu_sc as plsc

NC = 2
NS = 16
NW = NC * NS

CHUNK = 12800


@functools.partial(jax.jit, static_argnames=("total",))
def _gather_flat(xf, table, total):
    per_w = total // NW
    n_chunks = per_w // CHUNK
    mesh = plsc.VectorSubcoreMesh(core_axis_name="c", subcore_axis_name="s")

    @functools.partial(
        pl.kernel,
        out_type=jax.ShapeDtypeStruct((total,), jnp.float32),
        mesh=mesh,
        scratch_types=[
            pltpu.VMEM((CHUNK,), jnp.int32),
            pltpu.VMEM((CHUNK,), jnp.float32),
            pltpu.SemaphoreType.DMA,
        ],
    )
    def k(x_hbm, tab_hbm, out_hbm, idx_v, val_v, sem):
        wid = lax.axis_index("s") * NC + lax.axis_index("c")
        base = wid * per_w

        def chunk_body(c, carry):
            off = base + c * CHUNK
            pltpu.sync_copy(x_hbm.at[pl.ds(off, CHUNK)], idx_v)
            pltpu.async_copy(tab_hbm.at[idx_v], val_v, sem).wait()
            pltpu.sync_copy(val_v, out_hbm.at[pl.ds(off, CHUNK)])
            return carry

        lax.fori_loop(0, n_chunks, chunk_body, 0)

    return k(xf, table)


def kernel(x, vocab_table):
    shape = x.shape
    xf = x.reshape(-1).astype(jnp.int32)
    out = _gather_flat(xf, vocab_table, xf.size)
    return out.reshape(shape)


# speedup vs baseline: 1.2391x; 1.0500x over previous
import functools

import jax
import jax.numpy as jnp
from jax import lax
from jax.experimental import pallas as pl
from jax.experimental.pallas import tpu as pltpu
from jax.experimental.pallas import tpu_sc as plsc

NC = 2
NS = 16
NW = NC * NS

CHUNK = 12800


@functools.partial(jax.jit, static_argnames=("total",))
def _gather_flat(xf, table, total):
    per_w = total // NW
    n = per_w // CHUNK
    mesh = plsc.VectorSubcoreMesh(core_axis_name="c", subcore_axis_name="s")

    @functools.partial(
        pl.kernel,
        out_type=jax.ShapeDtypeStruct((total,), jnp.float32),
        mesh=mesh,
        scratch_types=[
            pltpu.VMEM((CHUNK,), jnp.int32),
            pltpu.VMEM((CHUNK,), jnp.int32),
            pltpu.VMEM((CHUNK,), jnp.float32),
            pltpu.VMEM((CHUNK,), jnp.float32),
            pltpu.SemaphoreType.DMA,
            pltpu.SemaphoreType.DMA,
            pltpu.SemaphoreType.DMA,
            pltpu.SemaphoreType.DMA,
            pltpu.SemaphoreType.DMA,
            pltpu.SemaphoreType.DMA,
        ],
    )
    def k(x_hbm, tab_hbm, out_hbm, i0, i1, v0, v1, si0, si1, sg0, sg1, so0, so1):
        wid = lax.axis_index("s") * NC + lax.axis_index("c")
        base = wid * per_w
        idx_v = [i0, i1]
        val_v = [v0, v1]
        sidx = [si0, si1]
        sgat = [sg0, sg1]
        sout = [so0, so1]

        def start_idx(c):
            return pltpu.async_copy(
                x_hbm.at[pl.ds(base + c * CHUNK, CHUNK)], idx_v[c % 2], sidx[c % 2]
            )

        def start_gat(c):
            return pltpu.async_copy(tab_hbm.at[idx_v[c % 2]], val_v[c % 2], sgat[c % 2])

        def start_out(c):
            return pltpu.async_copy(
                val_v[c % 2], out_hbm.at[pl.ds(base + c * CHUNK, CHUNK)], sout[c % 2]
            )

        idx_d, gat_d, out_d = {}, {}, {}
        idx_d[0] = start_idx(0)
        if n > 1:
            idx_d[1] = start_idx(1)
        idx_d[0].wait()
        gat_d[0] = start_gat(0)
        for c in range(n):
            if c + 1 < n:
                idx_d[c + 1].wait()
                if c >= 1:
                    out_d[c - 1].wait()
                gat_d[c + 1] = start_gat(c + 1)
            gat_d[c].wait()
            out_d[c] = start_out(c)
            if c + 2 < n:
                idx_d[c + 2] = start_idx(c + 2)
        if n >= 2:
            out_d[n - 2].wait()
        out_d[n - 1].wait()

    return k(xf, table)


def kernel(x, vocab_table):
    shape = x.shape
    xf = x.reshape(-1).astype(jnp.int32)
    out = _gather_flat(xf, vocab_table, xf.size)
    return out.reshape(shape)


# speedup vs baseline: 1.3582x; 1.0961x over previous
import functools

import jax
import jax.numpy as jnp
from jax import lax
from jax.experimental import pallas as pl
from jax.experimental.pallas import tpu as pltpu
from jax.experimental.pallas import tpu_sc as plsc

NC = 2
NS = 16
NW = NC * NS
LANES = 16

R = 32


@jax.jit
def _gather2d(x, table):
    n_rows, n_cols = x.shape
    rows_per_w = n_rows // NW
    n = rows_per_w // R
    cw = R * n_cols
    col_offs = list(range(0, n_cols - LANES + 1, LANES))
    if n_cols % LANES:
        col_offs.append(n_cols - LANES)
    mesh = plsc.VectorSubcoreMesh(core_axis_name="c", subcore_axis_name="s")

    @functools.partial(
        pl.kernel,
        out_type=jax.ShapeDtypeStruct((n_rows * n_cols,), jnp.float32),
        mesh=mesh,
        scratch_types=[
            pltpu.VMEM((R, n_cols), jnp.int32),
            pltpu.VMEM((R, n_cols), jnp.int32),
            pltpu.VMEM((cw,), jnp.int32),
            pltpu.VMEM((cw,), jnp.int32),
            pltpu.VMEM((cw,), jnp.float32),
            pltpu.VMEM((cw,), jnp.float32),
            pltpu.SemaphoreType.DMA,
            pltpu.SemaphoreType.DMA,
            pltpu.SemaphoreType.DMA,
            pltpu.SemaphoreType.DMA,
            pltpu.SemaphoreType.DMA,
            pltpu.SemaphoreType.DMA,
        ],
    )
    def k(x_hbm, tab_hbm, out_hbm,
          xa0, xa1, il0, il1, vl0, vl1,
          si0, si1, sg0, sg1, so0, so1):
        wid = lax.axis_index("s") * NC + lax.axis_index("c")
        base_row = wid * rows_per_w
        xa = [xa0, xa1]
        il = [il0, il1]
        vl = [vl0, vl1]
        sidx = [si0, si1]
        sgat = [sg0, sg1]
        sout = [so0, so1]

        def start_in(c):
            s = c % 2
            return pltpu.async_copy(
                x_hbm.at[pl.ds(base_row + c * R, R), :], xa[s], sidx[s]
            )

        def detile(c):
            s = c % 2

            def row(r, carry):
                for c0 in col_offs:
                    il[s][pl.ds(r * n_cols + c0, LANES)] = xa[s][r, pl.ds(c0, LANES)]
                return carry

            lax.fori_loop(0, R, row, 0)

        def start_gat(c):
            s = c % 2
            return pltpu.async_copy(tab_hbm.at[il[s]], vl[s], sgat[s])

        def start_out(c):
            s = c % 2
            return pltpu.async_copy(
                vl[s], out_hbm.at[pl.ds((base_row + c * R) * n_cols, cw)], sout[s]
            )

        in_d, gat_d, out_d = {}, {}, {}
        in_d[0] = start_in(0)
        if n > 1:
            in_d[1] = start_in(1)
        for c in range(n):
            in_d[c].wait()
            if c >= 2:
                out_d[c - 2].wait()
            detile(c)
            if c + 2 < n:
                in_d[c + 2] = start_in(c + 2)
            gat_d[c] = start_gat(c)
            if c >= 1:
                gat_d[c - 1].wait()
                out_d[c - 1] = start_out(c - 1)
        gat_d[n - 1].wait()
        out_d[n - 1] = start_out(n - 1)
        if n >= 2:
            out_d[n - 2].wait()
        out_d[n - 1].wait()

    return k(x, table)


def kernel(x, vocab_table):
    out = _gather2d(x.astype(jnp.int32), vocab_table)
    return out.reshape(x.shape)
